# Initial kernel scaffold; baseline (speedup 1.0000x reference)
import jax
import jax.numpy as jnp
from jax.experimental import pallas as pl


def kernel(z, embedding_weight):
    raise NotImplementedError("write your pallas kernel here")


# speedup vs baseline: 1.0849x; 1.0849x over previous
import jax
import jax.numpy as jnp
from jax.experimental import pallas as pl
from jax.experimental.pallas import tpu as pltpu

N_E = 1024
E_DIM = 64
BETA = 0.25
B = 16
P = 1024


def _vq_kernel(z_ref, e_ref, enc_ref, zq_ref, idx_ref, loss_ref, perp_ref,
               loss_acc, cnt_acc):
    b = pl.program_id(0)
    zb = z_ref[0]
    e = e_ref[...]
    zf = zb.T

    scores = jax.lax.dot_general(zf, e, (((1,), (1,)), ((), ())))
    z_sq = jnp.sum(zf * zf, axis=1, keepdims=True)
    e_sq = jnp.sum(e * e, axis=1)
    dist = (z_sq + e_sq[None, :]) - 2.0 * scores

    iota = jax.lax.broadcasted_iota(jnp.int32, (P, N_E), 1)
    m = jnp.min(dist, axis=1, keepdims=True)
    idx = jnp.min(jnp.where(dist == m, iota, N_E), axis=1).astype(jnp.int32)

    enc = (iota == idx[:, None]).astype(jnp.float32)
    enc_ref[0] = enc
    idx_ref[0, 0] = idx

    zq_t = jax.lax.dot_general(e, enc, (((0,), (1,)), ((), ())))
    diff = zq_t - zb
    zq_ref[0] = zb + diff

    @pl.when(b == 0)
    def _init():
        loss_acc[...] = jnp.zeros((1, 1), jnp.float32)
        cnt_acc[...] = jnp.zeros((1, N_E), jnp.float32)

    loss_acc[...] = loss_acc[...] + jnp.sum(diff * diff)
    cnt_acc[...] = cnt_acc[...] + jnp.sum(enc, axis=0)[None, :]

    @pl.when(b == B - 1)
    def _fin():
        loss_ref[...] = loss_acc[...] / (B * E_DIM * P) * BETA
        p = cnt_acc[...] / (B * P)
        perp_ref[...] = jnp.exp(-jnp.sum(p * jnp.log(p + 1e-10))).reshape(1, 1)


def kernel(z, embedding_weight):
    z3 = z.reshape(B, E_DIM, P)
    out_shapes = (
        jax.ShapeDtypeStruct((B, P, N_E), jnp.float32),
        jax.ShapeDtypeStruct((B, E_DIM, P), jnp.float32),
        jax.ShapeDtypeStruct((B, 1, P), jnp.int32),
        jax.ShapeDtypeStruct((1, 1), jnp.float32),
        jax.ShapeDtypeStruct((1, 1), jnp.float32),
    )
    enc, zq, idx, loss, perp = pl.pallas_call(
        _vq_kernel,
        grid=(B,),
        in_specs=[
            pl.BlockSpec((1, E_DIM, P), lambda b: (b, 0, 0)),
            pl.BlockSpec((N_E, E_DIM), lambda b: (0, 0)),
        ],
        out_specs=(
            pl.BlockSpec((1, P, N_E), lambda b: (b, 0, 0)),
            pl.BlockSpec((1, E_DIM, P), lambda b: (b, 0, 0)),
            pl.BlockSpec((1, 1, P), lambda b: (b, 0, 0)),
            pl.BlockSpec((1, 1), lambda b: (0, 0)),
            pl.BlockSpec((1, 1), lambda b: (0, 0)),
        ),
        out_shape=out_shapes,
        scratch_shapes=[
            pltpu.VMEM((1, 1), jnp.float32),
            pltpu.VMEM((1, N_E), jnp.float32),
        ],
    )(z3, embedding_weight)
    loss_s = loss[0, 0]
    perp_s = perp[0, 0]
    min_encodings = enc.reshape(B * P, N_E)
    min_encoding_indices = idx.reshape(B * P)
    z_q_st = zq.reshape(z.shape)
    return (loss_s, z_q_st, perp_s, min_encodings, min_encoding_indices)


# speedup vs baseline: 1.2415x; 1.1444x over previous
import jax
import jax.numpy as jnp
from jax.experimental import pallas as pl
from jax.experimental.pallas import tpu as pltpu

N_E = 1024
E_DIM = 64
BETA = 0.25
B = 16
P = 1024
N_TOK = B * P


def _vq_kernel(zf_ref, e_ref, enc_ref, zq_ref, idx_ref, loss_ref, cnt_ref):
    zf = zf_ref[...]
    e = e_ref[...]

    scores = jax.lax.dot_general(zf, e, (((1,), (1,)), ((), ())))
    z_sq = jnp.sum(zf * zf, axis=1, keepdims=True)
    e_sq = jnp.sum(e * e, axis=1)
    dist = (z_sq + e_sq[None, :]) - 2.0 * scores

    iota_f = jax.lax.broadcasted_iota(jnp.int32, (P, N_E), 1).astype(jnp.float32)
    m = jnp.min(dist, axis=1, keepdims=True)
    idx_f = jnp.min(jnp.where(dist == m, iota_f, float(N_E)), axis=1,
                    keepdims=True)

    enc = (iota_f == idx_f).astype(jnp.float32)
    enc_ref[0] = enc
    idx_ref[0, 0] = idx_f[:, 0].astype(jnp.int32)

    zq = jax.lax.dot_general(enc, e, (((1,), (0,)), ((), ())))
    diff = zq - zf
    zq_ref[...] = zf + diff

    loss_ref[...] = jnp.sum(diff * diff).reshape(1, 1, 1)
    cnt_ref[0] = jnp.sum(enc, axis=0, keepdims=True)


def kernel(z, embedding_weight):
    z_flat = jnp.transpose(z.reshape(B, E_DIM, P), (0, 2, 1)).reshape(N_TOK, E_DIM)
    out_shapes = (
        jax.ShapeDtypeStruct((B, P, N_E), jnp.float32),
        jax.ShapeDtypeStruct((N_TOK, E_DIM), jnp.float32),
        jax.ShapeDtypeStruct((B, 1, P), jnp.int32),
        jax.ShapeDtypeStruct((B, 1, 1), jnp.float32),
        jax.ShapeDtypeStruct((B, 1, N_E), jnp.float32),
    )
    enc, zq, idx, loss_p, cnt_p = pl.pallas_call(
        _vq_kernel,
        grid=(B,),
        in_specs=[
            pl.BlockSpec((P, E_DIM), lambda b: (b, 0)),
            pl.BlockSpec((N_E, E_DIM), lambda b: (0, 0)),
        ],
        out_specs=(
            pl.BlockSpec((1, P, N_E), lambda b: (b, 0, 0)),
            pl.BlockSpec((P, E_DIM), lambda b: (b, 0)),
            pl.BlockSpec((1, 1, P), lambda b: (b, 0, 0)),
            pl.BlockSpec((1, 1, 1), lambda b: (b, 0, 0)),
            pl.BlockSpec((1, 1, N_E), lambda b: (b, 0, 0)),
        ),
        out_shape=out_shapes,
        compiler_params=pltpu.CompilerParams(
            dimension_semantics=("parallel",),
        ),
    )(z_flat, embedding_weight)
    loss = jnp.sum(loss_p) / (N_TOK * E_DIM) * BETA
    avg_probs = jnp.sum(cnt_p[:, 0, :], axis=0) / N_TOK
    perplexity = jnp.exp(-jnp.sum(avg_probs * jnp.log(avg_probs + 1e-10)))
    min_encodings = enc.reshape(N_TOK, N_E)
    min_encoding_indices = idx.reshape(N_TOK)
    z_q_st = jnp.transpose(zq.reshape(B, P, E_DIM), (0, 2, 1)).reshape(z.shape)
    return (loss, z_q_st, perplexity, min_encodings, min_encoding_indices)
